# baseline (speedup 1.0000x reference)
import functools

import jax
import jax.numpy as jnp
import numpy as np
from jax import lax
from jax.experimental import pallas as pl
from jax.experimental.pallas import tpu as pltpu
from jax.experimental.pallas import tpu_sc as plsc

N = 10000
E = 320000
D_IN = 128
HID = 64
NUM_CLASSES = 40
NUM_LAYERS = 4
ALPHA = 0.1
THETA = 0.5

NC = 2
NS = 16
NW = NC * NS

CH = 128
K0 = 128
K1 = 32
CHUNKS = (K0 + K1) // 2
EP = NS * (K0 + K1) * CH
NPAD = 10240
ZROWS = NPAD // NS
RPT = NPAD // NS

_betas = [float(np.log(THETA / (l + 1) + 1.0)) for l in range(NUM_LAYERS)]


NBUF = 8


def _seg_sum_body(h_hbm, src0_hbm, dst0_hbm, src1_hbm, dst1_hbm, zeros_hbm,
                  out_hbm, si, di, rows_v, agg_sh, *sems):
    gsems = sems[:NBUF]
    ssems = sems[NBUF:2 * NBUF]
    isems = sems[2 * NBUF:]
    c = lax.axis_index("c")
    s = lax.axis_index("s")

    pltpu.sync_copy(zeros_hbm, agg_sh.at[pl.ds(s * ZROWS, ZROWS)])

    def run_loop(src_rows, dst_rows, n_chunks):
        rounds = n_chunks // NBUF

        def gather(pp, b):
            pltpu.async_copy(h_hbm.at[si.at[pp].at[b]], rows_v.at[b],
                             gsems[b])

        def gather_wait(b):
            pltpu.make_async_copy(h_hbm.at[si.at[0].at[0]], rows_v.at[b],
                                  gsems[b]).wait()

        def scatter(pp, b):
            pltpu.async_copy(rows_v.at[b], agg_sh.at[di.at[pp].at[b]],
                             ssems[b], add=True)

        def scatter_wait(b):
            pltpu.make_async_copy(rows_v.at[b], agg_sh.at[di.at[0].at[0]],
                                  ssems[b]).wait()

        def stage(t, pp):
            pltpu.async_copy(src_rows.at[pl.ds(t * NBUF, NBUF)], si.at[pp],
                             isems[pp])
            pltpu.async_copy(dst_rows.at[pl.ds(t * NBUF, NBUF)], di.at[pp],
                             isems[pp])

        def stage_wait(pp):
            pltpu.make_async_copy(src_rows.at[pl.ds(0, NBUF)], si.at[pp],
                                  isems[pp]).wait()
            pltpu.make_async_copy(dst_rows.at[pl.ds(0, NBUF)], di.at[pp],
                                  isems[pp]).wait()

        stage(0, 0)
        stage_wait(0)
        for b in range(NBUF):
            gather(0, b)
        stage(1, 1)

        def pair_step(u, carry):
            for p in (0, 1):
                t = 2 * u + p

                for b in range(NBUF):
                    gather_wait(b)
                    scatter(p, b)

                @pl.when(t < rounds - 1)
                def _prefetch():
                    stage_wait(1 - p)
                    for b in range(NBUF):
                        scatter_wait(b)
                        gather(1 - p, b)

                @pl.when(t < rounds - 2)
                def _stage_ahead():
                    stage(t + 2, p)

            return carry

        lax.fori_loop(0, rounds // 2, pair_step, 0)
        for b in range(NBUF):
            scatter_wait(b)

    @pl.when(c == 0)
    def _core0():
        run_loop(src0_hbm.at[s], dst0_hbm.at[s], K0)

    @pl.when(c == 1)
    def _core1():
        run_loop(src1_hbm.at[s], dst1_hbm.at[s], K1)

    plsc.subcore_barrier()

    pltpu.sync_copy(agg_sh.at[pl.ds(s * RPT, RPT)],
                    out_hbm.at[c].at[pl.ds(s * RPT, RPT)])


@functools.lru_cache(maxsize=None)
def _build_seg_sum():
    return pl.kernel(
        _seg_sum_body,
        out_type=jax.ShapeDtypeStruct((NC, NPAD, HID), jnp.float32),
        mesh=plsc.VectorSubcoreMesh(core_axis_name="c", subcore_axis_name="s"),
        scratch_types=[
            pltpu.VMEM((2, NBUF, CH), jnp.int32),
            pltpu.VMEM((2, NBUF, CH), jnp.int32),
            pltpu.VMEM((NBUF, CH, HID), jnp.float32),
            pltpu.VMEM_SHARED((NPAD, HID), jnp.float32),
        ] + [pltpu.SemaphoreType.DMA] * (2 * NBUF + 2),
        compiler_params=pltpu.CompilerParams(use_tc_tiling_on_sc=False),
    )


def _seg_sum(h, idx, zeros):
    return _build_seg_sum()(h, *idx, zeros)


BN = 400


def _in_proj_kernel(x_ref, w_ref, b_ref, o_ref):
    y = jnp.dot(x_ref[...], w_ref[...], preferred_element_type=jnp.float32)
    o_ref[...] = jnp.maximum(y + b_ref[...], 0.0)


def _combine_kernel(beta, p_ref, h0_ref, w_ref, o_ref):
    out = (1.0 - ALPHA) * (p_ref[0] + p_ref[1]) + ALPHA * h0_ref[...]
    y = jnp.dot(out, w_ref[...], preferred_element_type=jnp.float32)
    o_ref[...] = jnp.maximum((1.0 - beta) * out + beta * y, 0.0)


def _final_kernel(beta, p_ref, h0_ref, w_ref, wq_ref, r_ref, s_ref, b_ref,
                  o_ref):
    out = (1.0 - ALPHA) * (p_ref[0] + p_ref[1]) + ALPHA * h0_ref[...]
    y = jnp.dot(out, w_ref[...], preferred_element_type=jnp.float32)
    h = jnp.maximum((1.0 - beta) * out + beta * y, 0.0)
    g = jnp.dot(h, wq_ref[...], preferred_element_type=jnp.float32)
    hr = jnp.dot(h, r_ref[...], preferred_element_type=jnp.float32)
    logits = jnp.dot(g * hr, s_ref[...], preferred_element_type=jnp.float32)
    logits = logits + b_ref[...]
    m = jnp.max(logits, axis=-1, keepdims=True)
    lse = jnp.log(jnp.sum(jnp.exp(logits - m), axis=-1, keepdims=True)) + m
    o_ref[...] = logits - lse


def _row_grid_call(body, out_dim, extra_specs):
    grid = (N // BN,)
    return pl.pallas_call(
        body,
        grid=grid,
        in_specs=extra_specs,
        out_specs=pl.BlockSpec((BN, out_dim), lambda i: (i, 0)),
        out_shape=jax.ShapeDtypeStruct((N, out_dim), jnp.float32),
    )


def _full(shape):
    nd = len(shape)
    return pl.BlockSpec(shape, lambda i, _n=nd: (0,) * _n)


def kernel(x, edge_index, W0, b0, conv_W, W_out, b_out):
    src = edge_index[0]
    dst = edge_index[1]
    pad = EP - E
    n0 = NS * K0 * CH
    src_p = jnp.concatenate([src, jnp.zeros((pad,), jnp.int32)])
    dst_p = jnp.concatenate([dst, jnp.full((pad,), N, jnp.int32)])
    idx = (src_p[:n0].reshape(NS, K0, CH), dst_p[:n0].reshape(NS, K0, CH),
           src_p[n0:].reshape(NS, K1, CH), dst_p[n0:].reshape(NS, K1, CH))
    zeros = jnp.zeros((ZROWS, HID), jnp.float32)

    wq = W_out.reshape(HID, HID, NUM_CLASSES).reshape(HID, HID * NUM_CLASSES)
    r = jnp.repeat(jnp.eye(HID, dtype=jnp.float32), NUM_CLASSES, axis=1)
    s = jnp.tile(jnp.eye(NUM_CLASSES, dtype=jnp.float32), (HID, 1))

    h = _row_grid_call(
        _in_proj_kernel, HID,
        [pl.BlockSpec((BN, D_IN), lambda i: (i, 0)),
         _full((D_IN, HID)), _full((1, HID))],
    )(x, W0, b0.reshape(1, HID))
    h0 = h

    p_spec = pl.BlockSpec((NC, BN, HID), lambda i: (0, i, 0))
    h_spec = pl.BlockSpec((BN, HID), lambda i: (i, 0))

    for l in range(NUM_LAYERS - 1):
        partials = _seg_sum(h, idx, zeros)
        h = _row_grid_call(
            functools.partial(_combine_kernel, _betas[l]), HID,
            [p_spec, h_spec, _full((HID, HID))],
        )(partials, h0, conv_W[l])

    partials = _seg_sum(h, idx, zeros)
    out = _row_grid_call(
        functools.partial(_final_kernel, _betas[NUM_LAYERS - 1]), NUM_CLASSES,
        [p_spec, h_spec, _full((HID, HID)), _full((HID, HID * NUM_CLASSES)),
         _full((HID, HID * NUM_CLASSES)), _full((HID * NUM_CLASSES, NUM_CLASSES)),
         _full((1, NUM_CLASSES))],
    )(partials, h0, conv_W[NUM_LAYERS - 1], wq, r, s, b_out.reshape(1, NUM_CLASSES))
    return out


# speedup vs baseline: 1.0384x; 1.0384x over previous
import functools

import jax
import jax.numpy as jnp
import numpy as np
from jax import lax
from jax.experimental import pallas as pl
from jax.experimental.pallas import tpu as pltpu
from jax.experimental.pallas import tpu_sc as plsc

N = 10000
E = 320000
D_IN = 128
HID = 64
NUM_CLASSES = 40
NUM_LAYERS = 4
ALPHA = 0.1
THETA = 0.5

NC = 2
NS = 16
NW = NC * NS

CH = 128
K0 = 160
K1 = 0
CHUNKS = (K0 + K1) // 2
EP = NS * K0 * CH
NPAD = 10240
ZROWS = NPAD // NS
RPT = NPAD // NS

_betas = [float(np.log(THETA / (l + 1) + 1.0)) for l in range(NUM_LAYERS)]


NBUF = 8


def _seg_sum_body(h_hbm, src0_hbm, dst0_hbm, zeros_hbm,
                  out_hbm, si, di, rows_v, agg_sh, *sems):
    gsems = sems[:NBUF]
    ssems = sems[NBUF:2 * NBUF]
    isems = sems[2 * NBUF:]
    c = lax.axis_index("c")
    s = lax.axis_index("s")

    @pl.when(c == 0)
    def _zero():
        pltpu.sync_copy(zeros_hbm, agg_sh.at[pl.ds(s * ZROWS, ZROWS)])

    def run_loop(src_rows, dst_rows, n_chunks):
        rounds = n_chunks // NBUF

        def gather(pp, b):
            pltpu.async_copy(h_hbm.at[si.at[pp].at[b]], rows_v.at[b],
                             gsems[b])

        def gather_wait(b):
            pltpu.make_async_copy(h_hbm.at[si.at[0].at[0]], rows_v.at[b],
                                  gsems[b]).wait()

        def scatter(pp, b):
            pltpu.async_copy(rows_v.at[b], agg_sh.at[di.at[pp].at[b]],
                             ssems[b], add=True)

        def scatter_wait(b):
            pltpu.make_async_copy(rows_v.at[b], agg_sh.at[di.at[0].at[0]],
                                  ssems[b]).wait()

        def stage(t, pp):
            pltpu.async_copy(src_rows.at[pl.ds(t * NBUF, NBUF)], si.at[pp],
                             isems[pp])
            pltpu.async_copy(dst_rows.at[pl.ds(t * NBUF, NBUF)], di.at[pp],
                             isems[pp])

        def stage_wait(pp):
            pltpu.make_async_copy(src_rows.at[pl.ds(0, NBUF)], si.at[pp],
                                  isems[pp]).wait()
            pltpu.make_async_copy(dst_rows.at[pl.ds(0, NBUF)], di.at[pp],
                                  isems[pp]).wait()

        stage(0, 0)
        stage_wait(0)
        for b in range(NBUF):
            gather(0, b)
        stage(1, 1)

        def pair_step(u, carry):
            for p in (0, 1):
                t = 2 * u + p

                for b in range(NBUF):
                    gather_wait(b)
                    scatter(p, b)

                @pl.when(t < rounds - 1)
                def _prefetch():
                    stage_wait(1 - p)
                    for b in range(NBUF):
                        scatter_wait(b)
                        gather(1 - p, b)

                @pl.when(t < rounds - 2)
                def _stage_ahead():
                    stage(t + 2, p)

            return carry

        lax.fori_loop(0, rounds // 2, pair_step, 0)
        for b in range(NBUF):
            scatter_wait(b)

    @pl.when(c == 0)
    def _core0():
        run_loop(src0_hbm.at[s], dst0_hbm.at[s], K0)
        plsc.subcore_barrier()
        pltpu.sync_copy(agg_sh.at[pl.ds(s * RPT, RPT)],
                        out_hbm.at[pl.ds(s * RPT, RPT)])


@functools.lru_cache(maxsize=None)
def _build_seg_sum():
    return pl.kernel(
        _seg_sum_body,
        out_type=jax.ShapeDtypeStruct((NPAD, HID), jnp.float32),
        mesh=plsc.VectorSubcoreMesh(core_axis_name="c", subcore_axis_name="s"),
        scratch_types=[
            pltpu.VMEM((2, NBUF, CH), jnp.int32),
            pltpu.VMEM((2, NBUF, CH), jnp.int32),
            pltpu.VMEM((NBUF, CH, HID), jnp.float32),
            pltpu.VMEM_SHARED((NPAD, HID), jnp.float32),
        ] + [pltpu.SemaphoreType.DMA] * (2 * NBUF + 2),
        compiler_params=pltpu.CompilerParams(use_tc_tiling_on_sc=False),
    )


def _seg_sum(h, idx, zeros):
    return _build_seg_sum()(h, *idx, zeros)


BN = 400


def _in_proj_kernel(x_ref, w_ref, b_ref, o_ref):
    y = jnp.dot(x_ref[...], w_ref[...], preferred_element_type=jnp.float32)
    o_ref[...] = jnp.maximum(y + b_ref[...], 0.0)


def _combine_kernel(beta, p_ref, h0_ref, w_ref, o_ref):
    out = (1.0 - ALPHA) * p_ref[...] + ALPHA * h0_ref[...]
    y = jnp.dot(out, w_ref[...], preferred_element_type=jnp.float32)
    o_ref[...] = jnp.maximum((1.0 - beta) * out + beta * y, 0.0)


def _final_kernel(beta, p_ref, h0_ref, w_ref, wq_ref, r_ref, s_ref, b_ref,
                  o_ref):
    out = (1.0 - ALPHA) * p_ref[...] + ALPHA * h0_ref[...]
    y = jnp.dot(out, w_ref[...], preferred_element_type=jnp.float32)
    h = jnp.maximum((1.0 - beta) * out + beta * y, 0.0)
    g = jnp.dot(h, wq_ref[...], preferred_element_type=jnp.float32)
    hr = jnp.dot(h, r_ref[...], preferred_element_type=jnp.float32)
    logits = jnp.dot(g * hr, s_ref[...], preferred_element_type=jnp.float32)
    logits = logits + b_ref[...]
    m = jnp.max(logits, axis=-1, keepdims=True)
    lse = jnp.log(jnp.sum(jnp.exp(logits - m), axis=-1, keepdims=True)) + m
    o_ref[...] = logits - lse


def _row_grid_call(body, out_dim, extra_specs):
    grid = (N // BN,)
    return pl.pallas_call(
        body,
        grid=grid,
        in_specs=extra_specs,
        out_specs=pl.BlockSpec((BN, out_dim), lambda i: (i, 0)),
        out_shape=jax.ShapeDtypeStruct((N, out_dim), jnp.float32),
    )


def _full(shape):
    nd = len(shape)
    return pl.BlockSpec(shape, lambda i, _n=nd: (0,) * _n)


def kernel(x, edge_index, W0, b0, conv_W, W_out, b_out):
    src = edge_index[0]
    dst = edge_index[1]
    pad = EP - E
    src_p = jnp.concatenate([src, jnp.zeros((pad,), jnp.int32)])
    dst_p = jnp.concatenate([dst, jnp.full((pad,), N, jnp.int32)])
    idx = (src_p.reshape(NS, K0, CH), dst_p.reshape(NS, K0, CH))
    zeros = jnp.zeros((ZROWS, HID), jnp.float32)

    wq = W_out.reshape(HID, HID, NUM_CLASSES).reshape(HID, HID * NUM_CLASSES)
    r = jnp.repeat(jnp.eye(HID, dtype=jnp.float32), NUM_CLASSES, axis=1)
    s = jnp.tile(jnp.eye(NUM_CLASSES, dtype=jnp.float32), (HID, 1))

    h = _row_grid_call(
        _in_proj_kernel, HID,
        [pl.BlockSpec((BN, D_IN), lambda i: (i, 0)),
         _full((D_IN, HID)), _full((1, HID))],
    )(x, W0, b0.reshape(1, HID))
    h0 = h

    p_spec = pl.BlockSpec((BN, HID), lambda i: (i, 0))
    h_spec = pl.BlockSpec((BN, HID), lambda i: (i, 0))

    for l in range(NUM_LAYERS - 1):
        partials = _seg_sum(h, idx, zeros)
        h = _row_grid_call(
            functools.partial(_combine_kernel, _betas[l]), HID,
            [p_spec, h_spec, _full((HID, HID))],
        )(partials, h0, conv_W[l])

    partials = _seg_sum(h, idx, zeros)
    out = _row_grid_call(
        functools.partial(_final_kernel, _betas[NUM_LAYERS - 1]), NUM_CLASSES,
        [p_spec, h_spec, _full((HID, HID)), _full((HID, HID * NUM_CLASSES)),
         _full((HID, HID * NUM_CLASSES)), _full((HID * NUM_CLASSES, NUM_CLASSES)),
         _full((1, NUM_CLASSES))],
    )(partials, h0, conv_W[NUM_LAYERS - 1], wq, r, s, b_out.reshape(1, NUM_CLASSES))
    return out


# speedup vs baseline: 1.1980x; 1.1537x over previous
import functools

import jax
import jax.numpy as jnp
import numpy as np
from jax import lax
from jax.experimental import pallas as pl
from jax.experimental.pallas import tpu as pltpu
from jax.experimental.pallas import tpu_sc as plsc

N = 10000
E = 320000
D_IN = 128
HID = 64
NUM_CLASSES = 40
NUM_LAYERS = 4
ALPHA = 0.1
THETA = 0.5

NC = 2
NS = 16
NW = NC * NS

CH = 128
K0 = 128
K1 = 32
CHUNKS = (K0 + K1) // 2
EP = NS * (K0 + K1) * CH
NPAD = 10240
ZROWS = NPAD // NS
RPT = NPAD // NS

_betas = [float(np.log(THETA / (l + 1) + 1.0)) for l in range(NUM_LAYERS)]


NBUF = 4


def _seg_sum_body(h_hbm, src_hbm, dst_hbm, zeros_hbm,
                  out_hbm, si, di, rows_v, agg_sh, *sems):
    gsems = sems[:NBUF]
    ssems = sems[NBUF:2 * NBUF]
    c = lax.axis_index("c")
    s = lax.axis_index("s")

    pltpu.sync_copy(zeros_hbm, agg_sh.at[pl.ds(s * ZROWS, ZROWS)])

    row0 = jnp.where(c == 0, s * K0, NS * K0 + s * K1)
    rounds = jnp.where(c == 0, K0 // NBUF, K1 // NBUF)

    def gather(pp, b):
        pltpu.async_copy(h_hbm.at[si.at[pp].at[b]], rows_v.at[b], gsems[b])

    def gather_wait(b):
        pltpu.make_async_copy(h_hbm.at[si.at[0].at[0]], rows_v.at[b],
                              gsems[b]).wait()

    def scatter(pp, b):
        pltpu.async_copy(rows_v.at[b], agg_sh.at[di.at[pp].at[b]],
                         ssems[b], add=True)

    def scatter_wait(b):
        pltpu.make_async_copy(rows_v.at[b], agg_sh.at[di.at[0].at[0]],
                              ssems[b]).wait()

    def stage(t, pp):
        pltpu.sync_copy(src_hbm.at[pl.ds(row0 + t * NBUF, NBUF)], si.at[pp])
        pltpu.sync_copy(dst_hbm.at[pl.ds(row0 + t * NBUF, NBUF)], di.at[pp])

    stage(0, 0)
    for b in range(NBUF):
        gather(0, b)

    def pair_step(u, carry):
        for p in (0, 1):
            t = 2 * u + p

            @pl.when(t < rounds - 1)
            def _stage():
                stage(t + 1, 1 - p)

            for b in range(NBUF):
                gather_wait(b)
                scatter(p, b)

            @pl.when(t < rounds - 1)
            def _prefetch():
                for b in range(NBUF):
                    scatter_wait(b)
                    gather(1 - p, b)

        return carry

    lax.fori_loop(0, rounds // 2, pair_step, 0)
    for b in range(NBUF):
        scatter_wait(b)
    plsc.subcore_barrier()

    pltpu.sync_copy(agg_sh.at[pl.ds(s * RPT, RPT)],
                    out_hbm.at[c].at[pl.ds(s * RPT, RPT)])


@functools.lru_cache(maxsize=None)
def _build_seg_sum():
    return pl.kernel(
        _seg_sum_body,
        out_type=jax.ShapeDtypeStruct((NC, NPAD, HID), jnp.float32),
        mesh=plsc.VectorSubcoreMesh(core_axis_name="c", subcore_axis_name="s"),
        scratch_types=[
            pltpu.VMEM((2, NBUF, CH), jnp.int32),
            pltpu.VMEM((2, NBUF, CH), jnp.int32),
            pltpu.VMEM((NBUF, CH, HID), jnp.float32),
            pltpu.VMEM_SHARED((NPAD, HID), jnp.float32),
        ] + [pltpu.SemaphoreType.DMA] * (2 * NBUF),
        compiler_params=pltpu.CompilerParams(use_tc_tiling_on_sc=False),
    )


def _seg_sum(h, idx, zeros):
    return _build_seg_sum()(h, *idx, zeros)


BN = 400


def _in_proj_kernel(x_ref, w_ref, b_ref, o_ref):
    y = jnp.dot(x_ref[...], w_ref[...], preferred_element_type=jnp.float32)
    o_ref[...] = jnp.maximum(y + b_ref[...], 0.0)


def _combine_kernel(beta, p_ref, h0_ref, w_ref, o_ref):
    out = (1.0 - ALPHA) * (p_ref[0] + p_ref[1]) + ALPHA * h0_ref[...]
    y = jnp.dot(out, w_ref[...], preferred_element_type=jnp.float32)
    o_ref[...] = jnp.maximum((1.0 - beta) * out + beta * y, 0.0)


def _final_kernel(beta, p_ref, h0_ref, w_ref, wq_ref, r_ref, s_ref, b_ref,
                  o_ref):
    out = (1.0 - ALPHA) * (p_ref[0] + p_ref[1]) + ALPHA * h0_ref[...]
    y = jnp.dot(out, w_ref[...], preferred_element_type=jnp.float32)
    h = jnp.maximum((1.0 - beta) * out + beta * y, 0.0)
    g = jnp.dot(h, wq_ref[...], preferred_element_type=jnp.float32)
    hr = jnp.dot(h, r_ref[...], preferred_element_type=jnp.float32)
    logits = jnp.dot(g * hr, s_ref[...], preferred_element_type=jnp.float32)
    logits = logits + b_ref[...]
    m = jnp.max(logits, axis=-1, keepdims=True)
    lse = jnp.log(jnp.sum(jnp.exp(logits - m), axis=-1, keepdims=True)) + m
    o_ref[...] = logits - lse


def _row_grid_call(body, out_dim, extra_specs):
    grid = (N // BN,)
    return pl.pallas_call(
        body,
        grid=grid,
        in_specs=extra_specs,
        out_specs=pl.BlockSpec((BN, out_dim), lambda i: (i, 0)),
        out_shape=jax.ShapeDtypeStruct((N, out_dim), jnp.float32),
    )


def _full(shape):
    nd = len(shape)
    return pl.BlockSpec(shape, lambda i, _n=nd: (0,) * _n)


def kernel(x, edge_index, W0, b0, conv_W, W_out, b_out):
    src = edge_index[0]
    dst = edge_index[1]
    pad = EP - E
    src_p = jnp.concatenate([src, jnp.zeros((pad,), jnp.int32)])
    dst_p = jnp.concatenate([dst, jnp.full((pad,), N, jnp.int32)])
    idx = (src_p.reshape(EP // CH, CH), dst_p.reshape(EP // CH, CH))
    zeros = jnp.zeros((ZROWS, HID), jnp.float32)

    wq = W_out.reshape(HID, HID, NUM_CLASSES).reshape(HID, HID * NUM_CLASSES)
    r = jnp.repeat(jnp.eye(HID, dtype=jnp.float32), NUM_CLASSES, axis=1)
    s = jnp.tile(jnp.eye(NUM_CLASSES, dtype=jnp.float32), (HID, 1))

    h = _row_grid_call(
        _in_proj_kernel, HID,
        [pl.BlockSpec((BN, D_IN), lambda i: (i, 0)),
         _full((D_IN, HID)), _full((1, HID))],
    )(x, W0, b0.reshape(1, HID))
    h0 = h

    p_spec = pl.BlockSpec((NC, BN, HID), lambda i: (0, i, 0))
    h_spec = pl.BlockSpec((BN, HID), lambda i: (i, 0))

    for l in range(NUM_LAYERS - 1):
        partials = _seg_sum(h, idx, zeros)
        h = _row_grid_call(
            functools.partial(_combine_kernel, _betas[l]), HID,
            [p_spec, h_spec, _full((HID, HID))],
        )(partials, h0, conv_W[l])

    partials = _seg_sum(h, idx, zeros)
    out = _row_grid_call(
        functools.partial(_final_kernel, _betas[NUM_LAYERS - 1]), NUM_CLASSES,
        [p_spec, h_spec, _full((HID, HID)), _full((HID, HID * NUM_CLASSES)),
         _full((HID, HID * NUM_CLASSES)), _full((HID * NUM_CLASSES, NUM_CLASSES)),
         _full((1, NUM_CLASSES))],
    )(partials, h0, conv_W[NUM_LAYERS - 1], wq, r, s, b_out.reshape(1, NUM_CLASSES))
    return out
